# baseline (speedup 1.0000x reference)
import functools

import numpy as np
import jax
import jax.numpy as jnp
from jax import lax
from jax.experimental import pallas as pl
from jax.experimental.pallas import tpu as pltpu
from jax.experimental.pallas import tpu_sc as plsc

_WIN = 12
_SEQ = _WIN * _WIN
_P = _SEQ * _SEQ
_NH = 32
_TBL = (2 * _WIN - 1) ** 2
_NVEC = _P // 16
_UNROLL = 16


def _static_index() -> np.ndarray:
    coords = np.stack(np.meshgrid(np.arange(_WIN), np.arange(_WIN), indexing="ij"))
    flat = coords.reshape(2, -1)
    rel = (flat[:, :, None] - flat[:, None, :]).transpose(1, 2, 0)
    rel[:, :, 0] += _WIN - 1
    rel[:, :, 1] += _WIN - 1
    rel[:, :, 0] *= 2 * _WIN - 1
    return rel.sum(-1).reshape(-1).astype(np.int32)


_IDX32 = _static_index() * _NH


@functools.cache
def _build_sc_kernel():
    mesh = plsc.VectorSubcoreMesh(
        core_axis_name="c", subcore_axis_name="s", num_cores=1
    )

    @functools.partial(
        pl.kernel,
        mesh=mesh,
        out_type=jax.ShapeDtypeStruct((_NH * _P,), jnp.float32),
        compiler_params=pltpu.CompilerParams(needs_layout_passes=False),
        scratch_types=[
            pltpu.VMEM((_TBL * _NH,), jnp.float32),
            pltpu.VMEM((_P,), jnp.int32),
            pltpu.VMEM((_P,), jnp.float32),
            pltpu.VMEM((_P,), jnp.float32),
        ],
    )
    def _sc_bias_gather(table_hbm, idx_hbm, out_hbm, tbl_v, idx_v, row0_v, row1_v):
        w = lax.axis_index("s")
        pltpu.sync_copy(table_hbm, tbl_v)
        pltpu.sync_copy(idx_hbm, idx_v)

        @plsc.parallel_loop(0, _P, step=16, unroll=_UNROLL)
        def _gather_body(off):
            base = idx_v[pl.ds(off, 16)]
            row0_v[pl.ds(off, 16)] = plsc.load_gather(tbl_v, [base + w])
            row1_v[pl.ds(off, 16)] = plsc.load_gather(tbl_v, [base + (w + 16)])

        pltpu.sync_copy(row0_v, out_hbm.at[pl.ds(w * _P, _P)])
        pltpu.sync_copy(row1_v, out_hbm.at[pl.ds((w + 16) * _P, _P)])

    return _sc_bias_gather


def kernel(relative_position_bias_table):
    table_flat = relative_position_bias_table.reshape(-1)
    out = _build_sc_kernel()(table_flat, jnp.asarray(_IDX32))
    return out.reshape(_NH, _SEQ, _SEQ)


# speedup vs baseline: 1.5384x; 1.5384x over previous
import functools

import numpy as np
import jax
import jax.numpy as jnp
from jax import lax
from jax.experimental import pallas as pl
from jax.experimental.pallas import tpu as pltpu
from jax.experimental.pallas import tpu_sc as plsc

_WIN = 12
_SEQ = _WIN * _WIN
_P = _SEQ * _SEQ
_NH = 32
_TBL = (2 * _WIN - 1) ** 2
_COLPAD = 544
_VPR = _SEQ // 16


def _static_index() -> np.ndarray:
    coords = np.stack(np.meshgrid(np.arange(_WIN), np.arange(_WIN), indexing="ij"))
    flat = coords.reshape(2, -1)
    rel = (flat[:, :, None] - flat[:, None, :]).transpose(1, 2, 0)
    rel[:, :, 0] += _WIN - 1
    rel[:, :, 1] += _WIN - 1
    rel[:, :, 0] *= 2 * _WIN - 1
    return rel.sum(-1).reshape(-1).astype(np.int32)


_IDX = _static_index()


@functools.cache
def _build_sc_kernel():
    mesh = plsc.VectorSubcoreMesh(core_axis_name="c", subcore_axis_name="s")

    @functools.partial(
        pl.kernel,
        mesh=mesh,
        out_type=jax.ShapeDtypeStruct((_NH, _SEQ, _SEQ), jnp.float32),
        compiler_params=pltpu.CompilerParams(needs_layout_passes=False),
        scratch_types=[
            pltpu.VMEM((_TBL, _NH), jnp.float32),
            pltpu.VMEM((_COLPAD,), jnp.float32),
            pltpu.VMEM((_P,), jnp.int32),
            pltpu.VMEM((_SEQ, _SEQ), jnp.float32),
        ],
    )
    def _sc_bias_gather(table_hbm, idx_hbm, out_hbm, tbl_v, col_v, idx_v, slab_v):
        w = lax.axis_index("s") * mesh.num_cores + lax.axis_index("c")
        pltpu.sync_copy(table_hbm, tbl_v)
        pltpu.sync_copy(idx_hbm, idx_v)

        lanes = lax.iota(jnp.int32, 16)

        @plsc.parallel_loop(0, _COLPAD, step=16)
        def _col_body(r):
            rows = jnp.minimum(r + lanes, _TBL - 1)
            col_v[pl.ds(r, 16)] = plsc.load_gather(tbl_v, [rows, lanes * 0 + w])

        @plsc.parallel_loop(0, _SEQ, step=1, unroll=4)
        def _row_body(i):
            for u in range(_VPR):
                off = i * _SEQ + u * 16
                base = idx_v[pl.ds(off, 16)]
                slab_v[i, pl.ds(u * 16, 16)] = plsc.load_gather(col_v, [base])

        pltpu.sync_copy(slab_v, out_hbm.at[w])

    return _sc_bias_gather


def kernel(relative_position_bias_table):
    return _build_sc_kernel()(relative_position_bias_table, jnp.asarray(_IDX))


# speedup vs baseline: 2.1305x; 1.3849x over previous
import functools

import numpy as np
import jax
import jax.numpy as jnp
from jax import lax
from jax.experimental import pallas as pl
from jax.experimental.pallas import tpu as pltpu
from jax.experimental.pallas import tpu_sc as plsc

_WIN = 12
_SEQ = _WIN * _WIN
_NH = 32
_TBL = (2 * _WIN - 1) ** 2
_VPR = _SEQ // 16

_J = np.arange(_SEQ)
_D_TAB = ((_J // _WIN) * (2 * _WIN - 1) + _J % _WIN).astype(np.int32)
_S_TAB = np.zeros(_SEQ + 16, dtype=np.int32)
_S_TAB[:_SEQ] = ((_J // _WIN) + _WIN - 1) * (2 * _WIN - 1) + _J % _WIN + _WIN - 1


@functools.cache
def _build_sc_kernel():
    mesh = plsc.VectorSubcoreMesh(core_axis_name="c", subcore_axis_name="s")

    @functools.partial(
        pl.kernel,
        mesh=mesh,
        out_type=jax.ShapeDtypeStruct((_NH, _SEQ, _SEQ), jnp.float32),
        compiler_params=pltpu.CompilerParams(needs_layout_passes=False),
        scratch_types=[
            pltpu.VMEM((1, _TBL), jnp.float32),
            pltpu.VMEM((_SEQ,), jnp.int32),
            pltpu.VMEM((_SEQ + 16,), jnp.int32),
            pltpu.VMEM((_SEQ, _SEQ), jnp.float32),
        ],
    )
    def _sc_bias_gather(tablet_hbm, d_hbm, s_hbm, out_hbm, col_v, d_v, s_v, slab_v):
        w = lax.axis_index("s") * mesh.num_cores + lax.axis_index("c")
        pltpu.sync_copy(tablet_hbm.at[w], col_v)
        pltpu.sync_copy(d_hbm, d_v)
        pltpu.sync_copy(s_hbm, s_v)

        zeros = lax.iota(jnp.int32, 16) * 0
        d_regs = [d_v[pl.ds(u * 16, 16)] for u in range(_VPR)]

        @plsc.parallel_loop(0, _SEQ, step=1, unroll=4)
        def _row_body(i):
            s_i = s_v[pl.ds(i, 16)][0]
            for u in range(_VPR):
                idx = s_i - d_regs[u]
                slab_v[i, pl.ds(u * 16, 16)] = plsc.load_gather(col_v, [zeros, idx])

        pltpu.sync_copy(slab_v, out_hbm.at[w])

    return _sc_bias_gather


def kernel(relative_position_bias_table):
    table_t = relative_position_bias_table.T.reshape(_NH, 1, _TBL)
    return _build_sc_kernel()(table_t, jnp.asarray(_D_TAB), jnp.asarray(_S_TAB))


# speedup vs baseline: 2.3842x; 1.1190x over previous
import functools

import numpy as np
import jax
import jax.numpy as jnp
from jax import lax
from jax.experimental import pallas as pl
from jax.experimental.pallas import tpu as pltpu
from jax.experimental.pallas import tpu_sc as plsc

_WIN = 12
_SEQ = _WIN * _WIN
_NH = 32
_TBL = (2 * _WIN - 1) ** 2
_VPR = _SEQ // 16

_J = np.arange(_SEQ)
_D_TAB = ((_J // _WIN) * (2 * _WIN - 1) + _J % _WIN).astype(np.int32)


@functools.cache
def _build_sc_kernel():
    mesh = plsc.VectorSubcoreMesh(core_axis_name="c", subcore_axis_name="s")

    @functools.partial(
        pl.kernel,
        mesh=mesh,
        out_type=jax.ShapeDtypeStruct((_NH, _SEQ, _SEQ), jnp.float32),
        compiler_params=pltpu.CompilerParams(needs_layout_passes=False),
        scratch_types=[
            pltpu.VMEM((1, _TBL), jnp.float32),
            pltpu.VMEM((_SEQ, _SEQ), jnp.float32),
        ],
    )
    def _sc_bias_gather(tablet_hbm, out_hbm, col_v, slab_v):
        w = lax.axis_index("s") * mesh.num_cores + lax.axis_index("c")
        pltpu.sync_copy(tablet_hbm.at[w], col_v)

        lanes = lax.iota(jnp.int32, 16)
        zeros = lanes * 0
        d_regs = []
        for u in range(_VPR):
            j = lanes + u * 16
            jh = j // _WIN
            d_regs.append(jh * (2 * _WIN - 1) + (j - jh * _WIN))

        @plsc.parallel_loop(0, _SEQ, step=1, unroll=4)
        def _row_body(i):
            ih = i // _WIN
            s_i = (ih + _WIN - 1) * (2 * _WIN - 1) + (i - ih * _WIN) + _WIN - 1
            for u in range(_VPR):
                idx = s_i - d_regs[u]
                slab_v[i, pl.ds(u * 16, 16)] = plsc.load_gather(col_v, [zeros, idx])

        pltpu.sync_copy(slab_v, out_hbm.at[w])

    return _sc_bias_gather


def kernel(relative_position_bias_table):
    table_t = relative_position_bias_table.T.reshape(_NH, 1, _TBL)
    return _build_sc_kernel()(table_t)


# speedup vs baseline: 2.3885x; 1.0018x over previous
import functools

import numpy as np
import jax
import jax.numpy as jnp
from jax import lax
from jax.experimental import pallas as pl
from jax.experimental.pallas import tpu as pltpu
from jax.experimental.pallas import tpu_sc as plsc

_WIN = 12
_SEQ = _WIN * _WIN
_NH = 32
_TBL = (2 * _WIN - 1) ** 2
_VPR = _SEQ // 16

_J = np.arange(_SEQ)
_D_TAB = ((_J // _WIN) * (2 * _WIN - 1) + _J % _WIN).astype(np.int32)


@functools.cache
def _build_sc_kernel():
    mesh = plsc.VectorSubcoreMesh(core_axis_name="c", subcore_axis_name="s")

    @functools.partial(
        pl.kernel,
        mesh=mesh,
        out_type=jax.ShapeDtypeStruct((_NH, _SEQ, _SEQ), jnp.float32),
        compiler_params=pltpu.CompilerParams(needs_layout_passes=False),
        scratch_types=[
            pltpu.VMEM((1, _TBL), jnp.float32),
            pltpu.VMEM((_SEQ, _SEQ), jnp.float32),
            pltpu.SemaphoreType.DMA,
        ],
    )
    def _sc_bias_gather(tablet_hbm, out_hbm, col_v, slab_v, sem):
        w = lax.axis_index("s") * mesh.num_cores + lax.axis_index("c")
        pltpu.sync_copy(tablet_hbm.at[w], col_v)

        lanes = lax.iota(jnp.int32, 16)
        zeros = lanes * 0
        d_regs = []
        for u in range(_VPR):
            j = lanes + u * 16
            jh = j // _WIN
            d_regs.append(jh * (2 * _WIN - 1) + (j - jh * _WIN))

        half = _SEQ // 2

        def _gather_row(i):
            ih = i // _WIN
            s_i = (ih + _WIN - 1) * (2 * _WIN - 1) + (i - ih * _WIN) + _WIN - 1
            for u in range(_VPR):
                idx = s_i - d_regs[u]
                slab_v[i, pl.ds(u * 16, 16)] = plsc.load_gather(col_v, [zeros, idx])

        plsc.parallel_loop(0, half, step=1, unroll=4)(_gather_row)
        top = pltpu.async_copy(
            slab_v.at[pl.ds(0, half)], out_hbm.at[w, pl.ds(0, half)], sem
        )
        plsc.parallel_loop(half, _SEQ, step=1, unroll=4)(_gather_row)
        top.wait()
        pltpu.sync_copy(
            slab_v.at[pl.ds(half, half)], out_hbm.at[w, pl.ds(half, half)]
        )

    return _sc_bias_gather


def kernel(relative_position_bias_table):
    table_t = relative_position_bias_table.T.reshape(_NH, 1, _TBL)
    return _build_sc_kernel()(table_t)


# speedup vs baseline: 2.3920x; 1.0014x over previous
import functools

import jax
import jax.numpy as jnp
from jax import lax
from jax.experimental import pallas as pl
from jax.experimental.pallas import tpu as pltpu
from jax.experimental.pallas import tpu_sc as plsc

_WIN = 12
_SEQ = _WIN * _WIN
_NH = 32
_TBL = (2 * _WIN - 1) ** 2
_VPR = _SEQ // 16


@functools.cache
def _build_sc_kernel():
    mesh = plsc.VectorSubcoreMesh(core_axis_name="c", subcore_axis_name="s")

    @functools.partial(
        pl.kernel,
        mesh=mesh,
        out_type=jax.ShapeDtypeStruct((_NH, _SEQ, _SEQ), jnp.float32),
        compiler_params=pltpu.CompilerParams(needs_layout_passes=False),
        scratch_types=[
            pltpu.VMEM((1, _TBL), jnp.float32),
            pltpu.VMEM((_SEQ, _SEQ), jnp.float32),
            pltpu.SemaphoreType.DMA,
        ],
    )
    def _sc_bias_gather(tablet_hbm, out_hbm, col_v, slab_v, sem):
        w = lax.axis_index("s") * mesh.num_cores + lax.axis_index("c")
        pltpu.sync_copy(tablet_hbm.at[w], col_v)

        lanes = lax.iota(jnp.int32, 16)
        zeros = lanes * 0
        d_regs = []
        for u in range(_VPR):
            j = lanes + u * 16
            jh = j // _WIN
            d_regs.append(jh * (2 * _WIN - 1) + (j - jh * _WIN))

        half = _SEQ // 2

        def _gather_row(i):
            ih = i // _WIN
            s_i = (ih + _WIN - 1) * (2 * _WIN - 1) + (i - ih * _WIN) + _WIN - 1
            for u in range(_VPR):
                idx = s_i - d_regs[u]
                slab_v[i, pl.ds(u * 16, 16)] = plsc.load_gather(col_v, [zeros, idx])

        plsc.parallel_loop(0, half, step=1, unroll=4)(_gather_row)
        top = pltpu.async_copy(
            slab_v.at[pl.ds(0, half)], out_hbm.at[w, pl.ds(0, half)], sem
        )
        plsc.parallel_loop(half, _SEQ, step=1, unroll=4)(_gather_row)
        top.wait()
        pltpu.sync_copy(
            slab_v.at[pl.ds(half, half)], out_hbm.at[w, pl.ds(half, half)]
        )

    return _sc_bias_gather


def kernel(relative_position_bias_table):
    table_t = relative_position_bias_table.T.reshape(_NH, 1, _TBL)
    return _build_sc_kernel()(table_t)
